# baseline (speedup 1.0000x reference)
import functools

import jax
import jax.numpy as jnp
from jax import lax
from jax.experimental import pallas as pl
from jax.experimental.pallas import tpu as pltpu
from jax.experimental.pallas import tpu_sc as plsc


def _pick_chunk(ew: int, limit: int) -> int:
    for ck in range(min(ew, limit), 15, -1):
        if ew % ck == 0 and ck % 16 == 0:
            return ck
    return ew


def _zpack_tc(delta_z, t, n, rowlen):
    bn = 13312
    nb = rowlen // bn
    grid = (t // 2, nb)

    def zpk(ze_ref, zo_ref, out_ref):
        a = ze_ref[...].reshape(1, bn).astype(jnp.bfloat16)
        b = zo_ref[...].reshape(1, bn).astype(jnp.bfloat16)
        lo = lax.convert_element_type(
            lax.bitcast_convert_type(a, jnp.uint16), jnp.uint32)
        hi = lax.convert_element_type(
            lax.bitcast_convert_type(b, jnp.uint16), jnp.uint32)
        out_ref[...] = lax.bitcast_convert_type(
            lo | lax.shift_left(hi, jnp.uint32(16)), jnp.int32).reshape(bn)

    return pl.pallas_call(
        zpk,
        grid=grid,
        in_specs=[
            pl.BlockSpec((1, bn, 1), lambda p, i: (2 * p, i, 0)),
            pl.BlockSpec((1, bn, 1), lambda p, i: (2 * p + 1, i, 0)),
        ],
        out_specs=pl.BlockSpec((bn,), lambda p, i: (p * nb + i,)),
        out_shape=jax.ShapeDtypeStruct((t // 2 * rowlen,), jnp.int32),
    )(delta_z, delta_z)


def _rcpack_tc(ei, e):
    be = 131072
    grid = (pl.cdiv(e, be),)

    def rcpk(rc_ref, out_ref):
        x = rc_ref[...]
        r = x[0:1, :]
        c = x[1:2, :]
        out_ref[...] = (r | lax.shift_left(c, 16)).reshape(be)

    return pl.pallas_call(
        rcpk,
        grid=grid,
        in_specs=[pl.BlockSpec((2, be), lambda i: (0, i))],
        out_specs=pl.BlockSpec((be,), lambda i: (i,)),
        out_shape=jax.ShapeDtypeStruct((e,), jnp.int32),
    )(ei)


@functools.partial(jax.jit, static_argnames=("t", "n", "e"))
def _smoothness(delta_z, ei, w, *, t, n, e):
    rowlen = 4 * 13312
    zp = _zpack_tc(delta_z, t, n, rowlen)
    rcp = _rcpack_tc(ei, e)

    info = plsc.get_sparse_core_info()
    nw = info.num_cores * info.num_subcores
    ew = e // nw
    ck = _pick_chunk(ew, 4000)
    nchunks = ew // ck
    ngroups = ck // 16
    unroll = 8 if ngroups % 8 == 0 else (5 if ngroups % 5 == 0 else 1)

    mesh = plsc.VectorSubcoreMesh(core_axis_name="c", subcore_axis_name="s")

    @functools.partial(
        pl.kernel,
        mesh=mesh,
        compiler_params=pltpu.CompilerParams(needs_layout_passes=False),
        out_type=jax.ShapeDtypeStruct((nw, 16), jnp.float32),
        scratch_types=[
            pltpu.VMEM((n,), jnp.int32),
            pltpu.VMEM((n,), jnp.int32),
            pltpu.VMEM((ck,), jnp.int32),
            pltpu.VMEM((ck,), jnp.int32),
            pltpu.VMEM((ck,), jnp.float32),
            pltpu.VMEM((ck,), jnp.float32),
            pltpu.VMEM((16,), jnp.float32),
            pltpu.SemaphoreType.DMA,
            pltpu.SemaphoreType.DMA,
        ],
    )
    def body(zp_hbm, rcp_hbm, w_hbm, out_hbm, ztab0, ztab1, rcb0, rcb1,
             wb0, wb1, accv, sem0, sem1):
        cid = lax.axis_index("c")
        sid = lax.axis_index("s")
        wid = sid * info.num_cores + cid
        ebase = wid * ew
        sems = (sem0, sem1)
        rcbufs, wbufs = (rcb0, rcb1), (wb0, wb1)

        def fire(k, buf):
            base = ebase + k * ck
            sem = sems[buf]
            return (
                pltpu.async_copy(rcp_hbm.at[pl.ds(base, ck)], rcbufs[buf], sem),
                pltpu.async_copy(w_hbm.at[pl.ds(base, ck)], wbufs[buf], sem),
            )

        acc = jnp.zeros((16,), jnp.float32)

        def pass_body(q, acc):
            o0 = pl.multiple_of(2 * q * rowlen, 8)
            o1 = pl.multiple_of((2 * q + 1) * rowlen, 8)
            pltpu.sync_copy(zp_hbm.at[pl.ds(o0, n)], ztab0)
            pltpu.sync_copy(zp_hbm.at[pl.ds(o1, n)], ztab1)
            handles = fire(0, 0)
            for k in range(nchunks):
                cur = k % 2
                if k + 1 < nchunks:
                    next_handles = fire(k + 1, 1 - cur)
                for h in handles:
                    h.wait()
                if k + 1 < nchunks:
                    handles = next_handles

                @plsc.parallel_loop(0, ngroups, unroll=unroll, carry=acc)
                def group_loop(g, acc):
                    rcv = rcbufs[cur][pl.ds(g * 16, 16)]
                    wv = wbufs[cur][pl.ds(g * 16, 16)]
                    ri = rcv & 0xFFFF
                    ci = lax.shift_right_logical(rcv, 16)
                    s = jnp.zeros((16,), jnp.float32)
                    for ztab in (ztab0, ztab1):
                        aw = plsc.load_gather(ztab, [ri])
                        bw = plsc.load_gather(ztab, [ci])
                        a0, a1 = plsc.unpack(plsc.bitcast(aw, jnp.bfloat16),
                                             format=plsc.PackFormat.INTERLEAVED)
                        b0, b1 = plsc.unpack(plsc.bitcast(bw, jnp.bfloat16),
                                             format=plsc.PackFormat.INTERLEAVED)
                        d0 = a0 - b0
                        d1 = a1 - b1
                        s = s + (d0 * d0 + d1 * d1)
                    return acc + wv * s

                acc = group_loop
            return acc

        acc = lax.fori_loop(0, t // 4, pass_body, acc)

        accv[...] = acc
        pltpu.sync_copy(accv, out_hbm.at[wid])

    return body(zp, rcp, w)


def kernel(delta_z, edge_index, edge_weight):
    t, n, _ = delta_z.shape
    e = edge_weight.shape[0]
    ei = edge_index.astype(jnp.int32)
    partials = _smoothness(delta_z, ei, edge_weight, t=t, n=n, e=e)
    return partials.sum() / jnp.float32(t * e)


# speedup vs baseline: 3.4818x; 3.4818x over previous
import functools

import jax
import jax.numpy as jnp
from jax import lax
from jax.experimental import pallas as pl
from jax.experimental.pallas import tpu as pltpu
from jax.experimental.pallas import tpu_sc as plsc


def _pick_chunk(ew: int, limit: int) -> int:
    for ck in range(min(ew, limit), 15, -1):
        if ew % ck == 0 and ck % 16 == 0:
            return ck
    return ew


def _zpack_tc(z2, t, n, rowlen):
    bn = 13312
    nb = rowlen // bn
    grid = (nb, t // 2)

    def zpk(z_ref, out_ref):
        p = pl.program_id(1)
        a = z_ref[pl.ds(2 * p, 1), :].astype(jnp.bfloat16)
        b = z_ref[pl.ds(2 * p + 1, 1), :].astype(jnp.bfloat16)
        lo = lax.convert_element_type(
            lax.bitcast_convert_type(a, jnp.uint16), jnp.uint32)
        hi = lax.convert_element_type(
            lax.bitcast_convert_type(b, jnp.uint16), jnp.uint32)
        out_ref[...] = lax.bitcast_convert_type(
            lo | lax.shift_left(hi, jnp.uint32(16)), jnp.int32).reshape(bn)

    return pl.pallas_call(
        zpk,
        grid=grid,
        in_specs=[pl.BlockSpec((t, bn), lambda i, p: (0, i))],
        out_specs=pl.BlockSpec((bn,), lambda i, p: (p * nb + i,)),
        out_shape=jax.ShapeDtypeStruct((t // 2 * rowlen,), jnp.int32),
    )(z2)


def _rcpack_tc(ei, e):
    be = 131072
    grid = (pl.cdiv(e, be),)

    def rcpk(rc_ref, out_ref):
        x = rc_ref[...]
        r = x[0:1, :]
        c = x[1:2, :]
        out_ref[...] = (r | lax.shift_left(c, 16)).reshape(be)

    return pl.pallas_call(
        rcpk,
        grid=grid,
        in_specs=[pl.BlockSpec((2, be), lambda i: (0, i))],
        out_specs=pl.BlockSpec((be,), lambda i: (i,)),
        out_shape=jax.ShapeDtypeStruct((e,), jnp.int32),
    )(ei)


@functools.partial(jax.jit, static_argnames=("t", "n", "e"))
def _smoothness(delta_z, ei, w, *, t, n, e):
    rowlen = 4 * 13312
    zp = _zpack_tc(delta_z.reshape(t, n), t, n, rowlen)
    rcp = _rcpack_tc(ei, e)

    info = plsc.get_sparse_core_info()
    nw = info.num_cores * info.num_subcores
    ew = e // nw
    ck = _pick_chunk(ew, 4000)
    nchunks = ew // ck
    ngroups = ck // 16
    unroll = 8 if ngroups % 8 == 0 else (5 if ngroups % 5 == 0 else 1)

    mesh = plsc.VectorSubcoreMesh(core_axis_name="c", subcore_axis_name="s")

    @functools.partial(
        pl.kernel,
        mesh=mesh,
        compiler_params=pltpu.CompilerParams(needs_layout_passes=False),
        out_type=jax.ShapeDtypeStruct((nw, 16), jnp.float32),
        scratch_types=[
            pltpu.VMEM((n,), jnp.int32),
            pltpu.VMEM((n,), jnp.int32),
            pltpu.VMEM((ck,), jnp.int32),
            pltpu.VMEM((ck,), jnp.int32),
            pltpu.VMEM((ck,), jnp.float32),
            pltpu.VMEM((ck,), jnp.float32),
            pltpu.VMEM((16,), jnp.float32),
            pltpu.SemaphoreType.DMA,
            pltpu.SemaphoreType.DMA,
        ],
    )
    def body(zp_hbm, rcp_hbm, w_hbm, out_hbm, ztab0, ztab1, rcb0, rcb1,
             wb0, wb1, accv, sem0, sem1):
        cid = lax.axis_index("c")
        sid = lax.axis_index("s")
        wid = sid * info.num_cores + cid
        ebase = wid * ew
        sems = (sem0, sem1)
        rcbufs, wbufs = (rcb0, rcb1), (wb0, wb1)

        def fire(k, buf):
            base = ebase + k * ck
            sem = sems[buf]
            return (
                pltpu.async_copy(rcp_hbm.at[pl.ds(base, ck)], rcbufs[buf], sem),
                pltpu.async_copy(w_hbm.at[pl.ds(base, ck)], wbufs[buf], sem),
            )

        acc = jnp.zeros((16,), jnp.float32)

        def pass_body(q, acc):
            o0 = pl.multiple_of(2 * q * rowlen, 8)
            o1 = pl.multiple_of((2 * q + 1) * rowlen, 8)
            pltpu.sync_copy(zp_hbm.at[pl.ds(o0, n)], ztab0)
            pltpu.sync_copy(zp_hbm.at[pl.ds(o1, n)], ztab1)
            handles = fire(0, 0)
            for k in range(nchunks):
                cur = k % 2
                if k + 1 < nchunks:
                    next_handles = fire(k + 1, 1 - cur)
                for h in handles:
                    h.wait()
                if k + 1 < nchunks:
                    handles = next_handles

                @plsc.parallel_loop(0, ngroups, unroll=unroll, carry=acc)
                def group_loop(g, acc):
                    rcv = rcbufs[cur][pl.ds(g * 16, 16)]
                    wv = wbufs[cur][pl.ds(g * 16, 16)]
                    ri = rcv & 0xFFFF
                    ci = lax.shift_right_logical(rcv, 16)
                    s = jnp.zeros((16,), jnp.float32)
                    for ztab in (ztab0, ztab1):
                        aw = plsc.load_gather(ztab, [ri])
                        bw = plsc.load_gather(ztab, [ci])
                        a0, a1 = plsc.unpack(plsc.bitcast(aw, jnp.bfloat16),
                                             format=plsc.PackFormat.INTERLEAVED)
                        b0, b1 = plsc.unpack(plsc.bitcast(bw, jnp.bfloat16),
                                             format=plsc.PackFormat.INTERLEAVED)
                        d0 = a0 - b0
                        d1 = a1 - b1
                        s = s + (d0 * d0 + d1 * d1)
                    return acc + wv * s

                acc = group_loop
            return acc

        acc = lax.fori_loop(0, t // 4, pass_body, acc)

        accv[...] = acc
        pltpu.sync_copy(accv, out_hbm.at[wid])

    return body(zp, rcp, w)


def kernel(delta_z, edge_index, edge_weight):
    t, n, _ = delta_z.shape
    e = edge_weight.shape[0]
    ei = edge_index.astype(jnp.int32)
    partials = _smoothness(delta_z, ei, edge_weight, t=t, n=n, e=e)
    return partials.sum() / jnp.float32(t * e)
